# baseline (speedup 1.0000x reference)
import dataclasses

import jax
import jax.numpy as jnp
from jax import lax
from jax.experimental import pallas as pl
from jax.experimental.pallas import tpu as pltpu
from jax.experimental.pallas import tpu_sc as plsc

N = 10000
E = 160000
D = 256

NC = 2
NS = 16
NW = NC * NS
LANES = 16

RNG = 320
TRASH = RNG
ACC_ROWS = RNG + 8
ESC = 4000
NSUPER = E // ESC
B = 32
CAP = ESC + 2 * B


def _linear_body(x_ref, wt_ref, b_ref, o_ref):
    o_ref[...] = (
        jnp.dot(x_ref[...], wt_ref[...], preferred_element_type=jnp.float32)
        + b_ref[...]
    )


def _relu_body(a_ref, o_ref):
    o_ref[...] = jnp.maximum(a_ref[...], 0.0)


def _agg_body(h_hbm, src_hbm, dst_hbm, w_hbm, out_hbm,
              acc, src_v, dst_v, w_v, gidx_c, wc_c, dlc_c,
              msg0_v, msg1_v, sem0, sem1, sem2):
    cid = lax.axis_index("c")
    sid = lax.axis_index("s")
    wid = cid * NS + sid
    wlo = wid * RNG
    rng_u = jnp.minimum(RNG, N - wlo).astype(jnp.uint32)

    zvec = jnp.zeros((LANES,), jnp.float32)

    @plsc.parallel_loop(0, ACC_ROWS * D, step=8 * LANES, unroll=4)
    def _zacc(r):
        for j in range(0, 8 * LANES, LANES):
            acc[pl.ds(r + j, LANES)] = zvec

    lane = lax.iota(jnp.int32, LANES)

    def _issue_staging(s5):
        base = s5 * ESC
        pltpu.async_copy(src_hbm.at[pl.ds(base, ESC)], src_v, sem2)
        pltpu.async_copy(dst_hbm.at[pl.ds(base, ESC)], dst_v, sem2)
        pltpu.async_copy(w_hbm.at[pl.ds(base, ESC)], w_v, sem2)

    def _wait_staging():
        pltpu.make_async_copy(src_hbm.at[pl.ds(0, ESC)], src_v, sem2).wait()
        pltpu.make_async_copy(dst_hbm.at[pl.ds(0, ESC)], dst_v, sem2).wait()
        pltpu.make_async_copy(w_hbm.at[pl.ds(0, ESC)], w_v, sem2).wait()

    def _issue_gather(bb, msg_ref, sem):
        pltpu.async_copy(h_hbm.at[gidx_c.at[pl.ds(bb, B)]], msg_ref, sem)

    def _wait_gather(bb, msg_ref, sem):
        pltpu.make_async_copy(
            h_hbm.at[gidx_c.at[pl.ds(bb, B)]], msg_ref, sem).wait()

    def _mul_batch(bb, msg_ref):
        @plsc.parallel_loop(0, B, unroll=8)
        def _mul(k):
            e16 = jnp.full((LANES,), bb + k, jnp.int32)
            wb = plsc.load_gather(wc_c, [e16])
            row = plsc.load_gather(dlc_c, [e16])
            base = lax.shift_left(row, 8) + lane
            for j in range(0, D, LANES):
                val = msg_ref[k, pl.ds(j, LANES)] * wb
                plsc.addupdate_scatter(acc, [base + j], val)

    _issue_staging(0)

    @pl.loop(0, NSUPER)
    def _super(s5):
        _wait_staging()

        def _scan(j, offs):
            dvec = dst_v[pl.ds(j, LANES)]
            dloc = dvec - wlo
            m = plsc.bitcast(dloc, jnp.uint32) < rng_u
            cnt = plsc.all_reduce_population_count(m)
            pos = offs + plsc.cumsum(m.astype(jnp.int32))
            plsc.store_scatter(gidx_c, [pos], src_v[pl.ds(j, LANES)], mask=m)
            plsc.store_scatter(wc_c, [pos], w_v[pl.ds(j, LANES)], mask=m)
            plsc.store_scatter(dlc_c, [pos], dloc, mask=m)
            return offs + cnt

        offs = jnp.full((LANES,), -1, jnp.int32)
        offs = plsc.parallel_loop(
            0, ESC, step=LANES, unroll=16, carry=offs)(_scan)

        @pl.when(s5 + 1 < NSUPER)
        def _pref():
            _issue_staging(s5 + 1)

        k_cnt = jnp.max(offs) + 1
        k_rnd = (k_cnt + (2 * B - 1)) & ~(2 * B - 1)

        for t in range(0, 2 * B, LANES):
            pos = k_cnt + t + lane
            pm = pos < k_rnd
            plsc.store_scatter(gidx_c, [pos], wid + lane, mask=pm)
            plsc.store_scatter(wc_c, [pos], zvec, mask=pm)
            plsc.store_scatter(dlc_c, [pos],
                               jnp.full((LANES,), TRASH, jnp.int32), mask=pm)

        @pl.when(k_rnd > 0)
        def _flush():
            _issue_gather(0, msg0_v, sem0)

            @pl.loop(0, k_rnd, step=2 * B)
            def _pair(bb):
                _issue_gather(bb + B, msg1_v, sem1)
                _wait_gather(bb, msg0_v, sem0)
                _mul_batch(bb, msg0_v)

                @pl.when(bb + 2 * B < k_rnd)
                def _next():
                    _issue_gather(bb + 2 * B, msg0_v, sem0)

                _wait_gather(bb + B, msg1_v, sem1)
                _mul_batch(bb + B, msg1_v)

    @pl.when(wid < NW - 1)
    def _wb_full():
        pltpu.sync_copy(acc.at[pl.ds(0, RNG * D)],
                        out_hbm.at[pl.ds(wlo * D, RNG * D)])

    @pl.when(wid == NW - 1)
    def _wb_tail():
        pltpu.sync_copy(acc.at[pl.ds(0, 80 * D)],
                        out_hbm.at[pl.ds((NW - 1) * RNG * D, 80 * D)])


def _aggregate(h, src, dst, w):
    mesh = plsc.VectorSubcoreMesh(
        core_axis_name="c", subcore_axis_name="s",
        num_cores=NC, num_subcores=NS)
    cp = pltpu.CompilerParams()
    if "needs_layout_passes" in pltpu.CompilerParams.__dataclass_fields__:
        cp = dataclasses.replace(cp, needs_layout_passes=False)
    agg = pl.kernel(
        _agg_body,
        out_type=jax.ShapeDtypeStruct((N * D,), jnp.float32),
        mesh=mesh,
        scratch_types=[
            pltpu.VMEM((ACC_ROWS * D,), jnp.float32),
            pltpu.VMEM((ESC,), jnp.int32),
            pltpu.VMEM((ESC,), jnp.int32),
            pltpu.VMEM((ESC,), jnp.float32),
            pltpu.VMEM((CAP,), jnp.int32),
            pltpu.VMEM((CAP,), jnp.float32),
            pltpu.VMEM((CAP,), jnp.int32),
            pltpu.VMEM((B, D), jnp.float32),
            pltpu.VMEM((B, D), jnp.float32),
            pltpu.SemaphoreType.DMA,
            pltpu.SemaphoreType.DMA,
            pltpu.SemaphoreType.DMA,
        ],
        compiler_params=cp,
    )
    return agg(h, src, dst, w).reshape(N, D)


def kernel(x, edge_index, edge_weight, W, b):
    h = pl.pallas_call(
        _linear_body,
        grid=(N // 1000,),
        in_specs=[
            pl.BlockSpec((1000, D), lambda i: (i, 0)),
            pl.BlockSpec((D, D), lambda i: (0, 0)),
            pl.BlockSpec((1, D), lambda i: (0, 0)),
        ],
        out_specs=pl.BlockSpec((1000, D), lambda i: (i, 0)),
        out_shape=jax.ShapeDtypeStruct((N, D), jnp.float32),
    )(x, W.T, b.reshape(1, -1))

    raw = _aggregate(h, edge_index[0], edge_index[1], edge_weight)

    return pl.pallas_call(
        _relu_body,
        grid=(N // 1000,),
        in_specs=[pl.BlockSpec((1000, D), lambda i: (i, 0))],
        out_specs=pl.BlockSpec((1000, D), lambda i: (i, 0)),
        out_shape=jax.ShapeDtypeStruct((N, D), jnp.float32),
    )(raw)


# speedup vs baseline: 1.8735x; 1.8735x over previous
import dataclasses

import jax
import jax.numpy as jnp
from jax import lax
from jax.experimental import pallas as pl
from jax.experimental.pallas import tpu as pltpu
from jax.experimental.pallas import tpu_sc as plsc

N = 10000
E = 160000
D = 256

NC = 2
NS = 16
NW = NC * NS
LANES = 16

RNG = 320
TRASH = RNG
ACC_ROWS = RNG + 8
ESC = 4000
NSUPER = E // ESC
B = 32
CAP = ESC + 2 * B


def _linear_body(x_ref, wt_ref, b_ref, o_ref):
    o_ref[...] = (
        jnp.dot(x_ref[...], wt_ref[...], preferred_element_type=jnp.float32)
        + b_ref[...]
    )


def _relu_body(a_ref, o_ref):
    o_ref[...] = jnp.maximum(a_ref[...], 0.0)


def _agg_body(h_hbm, src_hbm, dst_hbm, w_hbm, out_hbm,
              acc, src_v, dst_v, w_v, gidx_c, wc_c, dlc_c,
              msg0_v, msg1_v, sem0, sem1, sem2):
    cid = lax.axis_index("c")
    sid = lax.axis_index("s")
    wid = cid * NS + sid
    wlo = wid * RNG
    rng_u = jnp.minimum(RNG, N - wlo).astype(jnp.uint32)

    zvec = jnp.zeros((LANES,), jnp.float32)

    @plsc.parallel_loop(0, ACC_ROWS * D, step=8 * LANES, unroll=4)
    def _zacc(r):
        for j in range(0, 8 * LANES, LANES):
            acc[pl.ds(r + j, LANES)] = zvec

    lane = lax.iota(jnp.int32, LANES)

    def _issue_staging(s5):
        base = s5 * ESC
        pltpu.async_copy(src_hbm.at[pl.ds(base, ESC)], src_v, sem2)
        pltpu.async_copy(dst_hbm.at[pl.ds(base, ESC)], dst_v, sem2)
        pltpu.async_copy(w_hbm.at[pl.ds(base, ESC)], w_v, sem2)

    def _wait_staging():
        pltpu.make_async_copy(src_hbm.at[pl.ds(0, ESC)], src_v, sem2).wait()
        pltpu.make_async_copy(dst_hbm.at[pl.ds(0, ESC)], dst_v, sem2).wait()
        pltpu.make_async_copy(w_hbm.at[pl.ds(0, ESC)], w_v, sem2).wait()

    def _issue_gather(bb, msg_ref, sem):
        pltpu.async_copy(h_hbm.at[gidx_c.at[pl.ds(bb, B)]], msg_ref, sem)

    def _wait_gather(bb, msg_ref, sem):
        pltpu.make_async_copy(
            h_hbm.at[gidx_c.at[pl.ds(bb, B)]], msg_ref, sem).wait()

    def _mul_batch(bb, msg_ref):
        @plsc.parallel_loop(0, B, unroll=4)
        def _mul(k):
            e16 = jnp.full((LANES,), bb + k, jnp.int32)
            wb = plsc.load_gather(wc_c, [e16])
            row = plsc.load_gather(dlc_c, [e16])
            base = lax.shift_left(row, 8) + lane
            for j in range(0, D, LANES):
                val = msg_ref[k, pl.ds(j, LANES)] * wb
                plsc.addupdate_scatter(acc, [base + j], val)

    _issue_staging(0)

    @pl.loop(0, NSUPER)
    def _super(s5):
        _wait_staging()

        def _scan(j, offs):
            dvec = dst_v[pl.ds(j, LANES)]
            dloc = dvec - wlo
            m = plsc.bitcast(dloc, jnp.uint32) < rng_u
            cnt = plsc.all_reduce_population_count(m)
            pos = offs + plsc.cumsum(m.astype(jnp.int32))
            plsc.store_scatter(gidx_c, [pos], src_v[pl.ds(j, LANES)], mask=m)
            plsc.store_scatter(wc_c, [pos], w_v[pl.ds(j, LANES)], mask=m)
            plsc.store_scatter(dlc_c, [pos], dloc, mask=m)
            return offs + cnt

        offs = jnp.full((LANES,), -1, jnp.int32)
        offs = plsc.parallel_loop(
            0, ESC, step=LANES, unroll=8, carry=offs)(_scan)

        @pl.when(s5 + 1 < NSUPER)
        def _pref():
            _issue_staging(s5 + 1)

        k_cnt = jnp.max(offs) + 1
        k_rnd = (k_cnt + (2 * B - 1)) & ~(2 * B - 1)

        for t in range(0, 2 * B, LANES):
            pos = k_cnt + t + lane
            pm = pos < k_rnd
            plsc.store_scatter(gidx_c, [pos], wid + lane, mask=pm)
            plsc.store_scatter(wc_c, [pos], zvec, mask=pm)
            plsc.store_scatter(dlc_c, [pos],
                               jnp.full((LANES,), TRASH, jnp.int32), mask=pm)

        @pl.when(k_rnd > 0)
        def _flush():
            _issue_gather(0, msg0_v, sem0)

            @pl.loop(0, k_rnd, step=2 * B)
            def _pair(bb):
                _issue_gather(bb + B, msg1_v, sem1)
                _wait_gather(bb, msg0_v, sem0)
                _mul_batch(bb, msg0_v)

                @pl.when(bb + 2 * B < k_rnd)
                def _next():
                    _issue_gather(bb + 2 * B, msg0_v, sem0)

                _wait_gather(bb + B, msg1_v, sem1)
                _mul_batch(bb + B, msg1_v)

    @plsc.parallel_loop(0, RNG * D, step=8 * LANES, unroll=4)
    def _relu(r):
        for j in range(0, 8 * LANES, LANES):
            acc[pl.ds(r + j, LANES)] = jnp.maximum(
                acc[pl.ds(r + j, LANES)], 0.0)

    @pl.when(wid < NW - 1)
    def _wb_full():
        pltpu.sync_copy(acc.at[pl.ds(0, RNG * D)],
                        out_hbm.at[pl.ds(wlo * D, RNG * D)])

    @pl.when(wid == NW - 1)
    def _wb_tail():
        pltpu.sync_copy(acc.at[pl.ds(0, 80 * D)],
                        out_hbm.at[pl.ds((NW - 1) * RNG * D, 80 * D)])


def _aggregate(h, src, dst, w):
    mesh = plsc.VectorSubcoreMesh(
        core_axis_name="c", subcore_axis_name="s",
        num_cores=NC, num_subcores=NS)
    cp = pltpu.CompilerParams()
    if "needs_layout_passes" in pltpu.CompilerParams.__dataclass_fields__:
        cp = dataclasses.replace(cp, needs_layout_passes=False)
    agg = pl.kernel(
        _agg_body,
        out_type=jax.ShapeDtypeStruct((N * D,), jnp.float32),
        mesh=mesh,
        scratch_types=[
            pltpu.VMEM((ACC_ROWS * D,), jnp.float32),
            pltpu.VMEM((ESC,), jnp.int32),
            pltpu.VMEM((ESC,), jnp.int32),
            pltpu.VMEM((ESC,), jnp.float32),
            pltpu.VMEM((CAP,), jnp.int32),
            pltpu.VMEM((CAP,), jnp.float32),
            pltpu.VMEM((CAP,), jnp.int32),
            pltpu.VMEM((B, D), jnp.float32),
            pltpu.VMEM((B, D), jnp.float32),
            pltpu.SemaphoreType.DMA,
            pltpu.SemaphoreType.DMA,
            pltpu.SemaphoreType.DMA,
        ],
        compiler_params=cp,
    )
    return agg(h, src, dst, w).reshape(N, D)


def kernel(x, edge_index, edge_weight, W, b):
    h = pl.pallas_call(
        _linear_body,
        grid=(N // 1000,),
        in_specs=[
            pl.BlockSpec((1000, D), lambda i: (i, 0)),
            pl.BlockSpec((D, D), lambda i: (0, 0)),
            pl.BlockSpec((1, D), lambda i: (0, 0)),
        ],
        out_specs=pl.BlockSpec((1000, D), lambda i: (i, 0)),
        out_shape=jax.ShapeDtypeStruct((N, D), jnp.float32),
    )(x, W.T, b.reshape(1, -1))

    return _aggregate(h, edge_index[0], edge_index[1], edge_weight)
